# baseline (speedup 1.0000x reference)
import functools

import jax
import jax.numpy as jnp
from jax.experimental import pallas as pl
from jax.experimental.pallas import tpu as pltpu

_Z_CELL = 0.1
_Z_HID = 0.1
_TS = 16


def _cell(xw, h, c, hid):
    gi = jax.nn.sigmoid(xw[:, :hid])
    gf = jax.nn.sigmoid(xw[:, hid : 2 * hid])
    gg = jnp.tanh(xw[:, 2 * hid : 3 * hid])
    go = jax.nn.sigmoid(xw[:, 3 * hid :])
    c_new = gf * c + gi * gg
    h_new = go * jnp.tanh(c_new)
    c_out = (1.0 - _Z_CELL) * c_new + _Z_CELL * c
    h_out = (1.0 - _Z_HID) * h_new + _Z_HID * h
    return h_out, c_out


def _phase(
    xw_rd, xw_wr, xf_ref, xb_ref, wih_ref, whh_ref, bias_ref, of_ref, ob_ref,
    h_ref, c_ref, ts, hid, nb,
):
    rows = ts * nb

    def _step(j):
        jb = ts - 1 - j
        h = h_ref[...]
        gates = jnp.dot(
            h.astype(jnp.bfloat16), whh_ref[...], preferred_element_type=jnp.float32
        )
        hf, cf = _cell(
            gates[:nb] + xw_rd[j * nb : (j + 1) * nb], h[:nb], c_ref[:nb], hid
        )
        hb, cb = _cell(
            gates[nb:] + xw_rd[rows + jb * nb : rows + (jb + 1) * nb],
            h[nb:],
            c_ref[nb:],
            hid,
        )
        h_ref[:nb] = hf
        h_ref[nb:] = hb
        c_ref[:nb] = cf
        c_ref[nb:] = cb
        of_ref[j] = hf
        ob_ref[jb] = hb

    _step(0)

    xcat = jnp.concatenate(
        [xf_ref[...].reshape(rows, -1), xb_ref[...].reshape(rows, -1)], axis=0
    ).astype(jnp.bfloat16)
    xw_wr[...] = (
        jnp.dot(xcat, wih_ref[...], preferred_element_type=jnp.float32)
        + bias_ref[...]
    )

    for j in range(1, ts):
        _step(j)


def _fused_kernel(
    xf_ref, xb_ref, wih_ref, whh_ref, bias_ref, of_ref, ob_ref,
    xw0_ref, xw1_ref, h_ref, c_ref, *, ts, hid, nb,
):
    t = pl.program_id(0)

    @pl.when(t == 1)
    def _():
        h_ref[...] = jnp.zeros_like(h_ref)
        c_ref[...] = jnp.zeros_like(c_ref)

    args = (
        xf_ref, xb_ref, wih_ref, whh_ref, bias_ref, of_ref, ob_ref,
        h_ref, c_ref, ts, hid, nb,
    )
    par = jax.lax.rem(t, 2)

    @pl.when(par == 0)
    def _():
        _phase(xw1_ref, xw0_ref, *args)

    @pl.when(par == 1)
    def _():
        _phase(xw0_ref, xw1_ref, *args)


def kernel(inputs, W_ih, W_hh, b_ih, b_hh):
    T, B, I = inputs.shape
    G = W_ih.shape[0]
    hid = G // 4
    ts = _TS
    nt = T // ts

    bias = (b_ih + b_hh).reshape(1, G)

    out_f, out_b = pl.pallas_call(
        functools.partial(_fused_kernel, ts=ts, hid=hid, nb=B),
        grid=(nt + 1,),
        in_specs=[
            pl.BlockSpec((ts, B, I), lambda t: (jnp.minimum(t, nt - 1), 0, 0)),
            pl.BlockSpec((ts, B, I), lambda t: (jnp.maximum(nt - 1 - t, 0), 0, 0)),
            pl.BlockSpec((I, G), lambda t: (0, 0)),
            pl.BlockSpec((hid, G), lambda t: (0, 0)),
            pl.BlockSpec((1, G), lambda t: (0, 0)),
        ],
        out_specs=[
            pl.BlockSpec((ts, B, hid), lambda t: (jnp.maximum(t - 1, 0), 0, 0)),
            pl.BlockSpec((ts, B, hid), lambda t: (jnp.minimum(nt - t, nt - 1), 0, 0)),
        ],
        out_shape=[
            jax.ShapeDtypeStruct((T, B, hid), jnp.float32),
            jax.ShapeDtypeStruct((T, B, hid), jnp.float32),
        ],
        scratch_shapes=[
            pltpu.VMEM((2 * ts * B, G), jnp.float32),
            pltpu.VMEM((2 * ts * B, G), jnp.float32),
            pltpu.VMEM((2 * B, hid), jnp.float32),
            pltpu.VMEM((2 * B, hid), jnp.float32),
        ],
        compiler_params=pltpu.CompilerParams(
            dimension_semantics=("arbitrary",),
            vmem_limit_bytes=48 * 1024 * 1024,
        ),
        name="lstm_fused",
    )(
        inputs,
        inputs,
        W_ih.T.astype(jnp.bfloat16),
        W_hh.T.astype(jnp.bfloat16),
        bias,
    )

    return out_f + out_b


# speedup vs baseline: 1.0775x; 1.0775x over previous
import functools

import jax
import jax.numpy as jnp
from jax.experimental import pallas as pl
from jax.experimental.pallas import tpu as pltpu

_Z_CELL = 0.1
_Z_HID = 0.1
_TS = 16


def _proj_kernel(x_ref, w_ref, b_ref, o_ref):
    o_ref[...] = (
        jnp.dot(x_ref[...], w_ref[...], preferred_element_type=jnp.float32)
        + b_ref[...]
    ).astype(jnp.bfloat16)


def _cell(xw, h, c, hid):
    gi = jax.nn.sigmoid(xw[:, :hid])
    gf = jax.nn.sigmoid(xw[:, hid : 2 * hid])
    gg = jnp.tanh(xw[:, 2 * hid : 3 * hid])
    go = jax.nn.sigmoid(xw[:, 3 * hid :])
    c_new = gf * c + gi * gg
    h_new = go * jnp.tanh(c_new)
    c_out = (1.0 - _Z_CELL) * c_new + _Z_CELL * c
    h_out = (1.0 - _Z_HID) * h_new + _Z_HID * h
    return h_out, c_out


def _lstm_kernel(xwf_ref, xwb_ref, whh_ref, of_ref, ob_ref, h_ref, c_ref, *, ts, hid, nb):
    @pl.when(pl.program_id(0) == 0)
    def _():
        h_ref[...] = jnp.zeros_like(h_ref)
        c_ref[...] = jnp.zeros_like(c_ref)

    for j in range(ts):
        jb = ts - 1 - j
        h = h_ref[...]
        gates = jnp.dot(
            h.astype(jnp.bfloat16), whh_ref[...], preferred_element_type=jnp.float32
        )
        hf, cf = _cell(
            gates[:nb] + xwf_ref[j].astype(jnp.float32), h[:nb], c_ref[:nb], hid
        )
        hb, cb = _cell(
            gates[nb:] + xwb_ref[jb].astype(jnp.float32), h[nb:], c_ref[nb:], hid
        )
        h_ref[:nb] = hf
        h_ref[nb:] = hb
        c_ref[:nb] = cf
        c_ref[nb:] = cb
        of_ref[j] = hf
        ob_ref[jb] = hb


def kernel(inputs, W_ih, W_hh, b_ih, b_hh):
    T, B, I = inputs.shape
    G = W_ih.shape[0]
    hid = G // 4
    ts = _TS
    nt = T // ts

    x2d = inputs.reshape(T * B, I)
    bias = (b_ih + b_hh).reshape(1, G)
    bm = 1024
    xw = pl.pallas_call(
        _proj_kernel,
        grid=((T * B) // bm,),
        in_specs=[
            pl.BlockSpec((bm, I), lambda m: (m, 0)),
            pl.BlockSpec((I, G), lambda m: (0, 0)),
            pl.BlockSpec((1, G), lambda m: (0, 0)),
        ],
        out_specs=pl.BlockSpec((bm, G), lambda m: (m, 0)),
        out_shape=jax.ShapeDtypeStruct((T * B, G), jnp.bfloat16),
        compiler_params=pltpu.CompilerParams(
            dimension_semantics=("arbitrary",),
            vmem_limit_bytes=48 * 1024 * 1024,
        ),
        name="lstm_in_proj",
    )(x2d, W_ih.T, bias).reshape(T, B, G)

    out_f, out_b = pl.pallas_call(
        functools.partial(_lstm_kernel, ts=ts, hid=hid, nb=B),
        grid=(nt,),
        in_specs=[
            pl.BlockSpec((ts, B, G), lambda t: (t, 0, 0)),
            pl.BlockSpec((ts, B, G), lambda t: (nt - 1 - t, 0, 0)),
            pl.BlockSpec((hid, G), lambda t: (0, 0)),
        ],
        out_specs=[
            pl.BlockSpec((ts, B, hid), lambda t: (t, 0, 0)),
            pl.BlockSpec((ts, B, hid), lambda t: (nt - 1 - t, 0, 0)),
        ],
        out_shape=[
            jax.ShapeDtypeStruct((T, B, hid), jnp.float32),
            jax.ShapeDtypeStruct((T, B, hid), jnp.float32),
        ],
        scratch_shapes=[
            pltpu.VMEM((2 * B, hid), jnp.float32),
            pltpu.VMEM((2 * B, hid), jnp.float32),
        ],
        compiler_params=pltpu.CompilerParams(
            dimension_semantics=("arbitrary",),
            vmem_limit_bytes=48 * 1024 * 1024,
        ),
        name="lstm_recurrence",
    )(xw, xw, W_hh.T.astype(jnp.bfloat16))

    return out_f + out_b


# speedup vs baseline: 1.0936x; 1.0149x over previous
import functools

import jax
import jax.numpy as jnp
from jax.experimental import pallas as pl
from jax.experimental.pallas import tpu as pltpu

_Z_CELL = 0.1
_Z_HID = 0.1
_TS = 32


def _sigmoid(x):
    return 0.5 + 0.5 * jnp.tanh(0.5 * x)


def _proj_kernel(x_ref, w_ref, b_ref, o_ref):
    o_ref[...] = (
        jnp.dot(x_ref[...], w_ref[...], preferred_element_type=jnp.float32)
        + b_ref[...]
    ).astype(jnp.bfloat16)


def _cell(xw, h, c, hid):
    gi = _sigmoid(xw[:, :hid])
    gf = _sigmoid(xw[:, hid : 2 * hid])
    gg = jnp.tanh(xw[:, 2 * hid : 3 * hid])
    go = _sigmoid(xw[:, 3 * hid :])
    c_new = gf * c + gi * gg
    h_new = go * jnp.tanh(c_new)
    c_out = (1.0 - _Z_CELL) * c_new + _Z_CELL * c
    h_out = (1.0 - _Z_HID) * h_new + _Z_HID * h
    return h_out, c_out


def _lstm_kernel(xwf_ref, xwb_ref, whh_ref, of_ref, ob_ref, h_ref, c_ref, *, ts, hid, nb):
    @pl.when(pl.program_id(0) == 0)
    def _():
        h_ref[...] = jnp.zeros_like(h_ref)
        c_ref[...] = jnp.zeros_like(c_ref)

    for j in range(ts):
        jb = ts - 1 - j
        h = h_ref[...]
        gates = jnp.dot(
            h.astype(jnp.bfloat16), whh_ref[...], preferred_element_type=jnp.float32
        )
        hf, cf = _cell(
            gates[:nb] + xwf_ref[j].astype(jnp.float32), h[:nb], c_ref[:nb], hid
        )
        hb, cb = _cell(
            gates[nb:] + xwb_ref[jb].astype(jnp.float32), h[nb:], c_ref[nb:], hid
        )
        h_ref[:nb] = hf
        h_ref[nb:] = hb
        c_ref[:nb] = cf
        c_ref[nb:] = cb
        of_ref[j] = hf
        ob_ref[jb] = hb


def kernel(inputs, W_ih, W_hh, b_ih, b_hh):
    T, B, I = inputs.shape
    G = W_ih.shape[0]
    hid = G // 4
    ts = _TS
    nt = T // ts

    x2d = inputs.reshape(T * B, I)
    bias = (b_ih + b_hh).reshape(1, G)
    bm = 1024
    xw = pl.pallas_call(
        _proj_kernel,
        grid=((T * B) // bm,),
        in_specs=[
            pl.BlockSpec((bm, I), lambda m: (m, 0)),
            pl.BlockSpec((I, G), lambda m: (0, 0)),
            pl.BlockSpec((1, G), lambda m: (0, 0)),
        ],
        out_specs=pl.BlockSpec((bm, G), lambda m: (m, 0)),
        out_shape=jax.ShapeDtypeStruct((T * B, G), jnp.bfloat16),
        compiler_params=pltpu.CompilerParams(
            dimension_semantics=("arbitrary",),
            vmem_limit_bytes=48 * 1024 * 1024,
        ),
        name="lstm_in_proj",
    )(x2d, W_ih.T, bias).reshape(T, B, G)

    out_f, out_b = pl.pallas_call(
        functools.partial(_lstm_kernel, ts=ts, hid=hid, nb=B),
        grid=(nt,),
        in_specs=[
            pl.BlockSpec((ts, B, G), lambda t: (t, 0, 0)),
            pl.BlockSpec((ts, B, G), lambda t: (nt - 1 - t, 0, 0)),
            pl.BlockSpec((hid, G), lambda t: (0, 0)),
        ],
        out_specs=[
            pl.BlockSpec((ts, B, hid), lambda t: (t, 0, 0)),
            pl.BlockSpec((ts, B, hid), lambda t: (nt - 1 - t, 0, 0)),
        ],
        out_shape=[
            jax.ShapeDtypeStruct((T, B, hid), jnp.float32),
            jax.ShapeDtypeStruct((T, B, hid), jnp.float32),
        ],
        scratch_shapes=[
            pltpu.VMEM((2 * B, hid), jnp.float32),
            pltpu.VMEM((2 * B, hid), jnp.float32),
        ],
        compiler_params=pltpu.CompilerParams(
            dimension_semantics=("arbitrary",),
            vmem_limit_bytes=48 * 1024 * 1024,
        ),
        name="lstm_recurrence",
    )(xw, xw, W_hh.T.astype(jnp.bfloat16))

    return out_f + out_b


# speedup vs baseline: 1.1066x; 1.0119x over previous
import functools

import jax
import jax.numpy as jnp
from jax.experimental import pallas as pl
from jax.experimental.pallas import tpu as pltpu

_Z_CELL = 0.1
_Z_HID = 0.1
_TS = 32


def _sigmoid(x):
    return 0.5 + 0.5 * jnp.tanh(0.5 * x)


def _proj_kernel(x_ref, w_ref, b_ref, o_ref):
    o_ref[...] = (
        jnp.dot(
            x_ref[...].astype(jnp.bfloat16),
            w_ref[...],
            preferred_element_type=jnp.float32,
        )
        + b_ref[...]
    ).astype(jnp.bfloat16)


def _cell(xw, h, c, hid):
    gi = _sigmoid(xw[:, :hid])
    gf = _sigmoid(xw[:, hid : 2 * hid])
    gg = jnp.tanh(xw[:, 2 * hid : 3 * hid])
    go = _sigmoid(xw[:, 3 * hid :])
    c_new = gf * c + gi * gg
    h_new = go * jnp.tanh(c_new)
    c_out = (1.0 - _Z_CELL) * c_new + _Z_CELL * c
    h_out = (1.0 - _Z_HID) * h_new + _Z_HID * h
    return h_out, c_out


def _lstm_kernel(xwf_ref, xwb_ref, whh_ref, of_ref, ob_ref, h_ref, c_ref, *, ts, hid, nb):
    @pl.when(pl.program_id(0) == 0)
    def _():
        h_ref[...] = jnp.zeros_like(h_ref)
        c_ref[...] = jnp.zeros_like(c_ref)

    for j in range(ts):
        jb = ts - 1 - j
        h = h_ref[...]
        gates = jnp.dot(
            h.astype(jnp.bfloat16), whh_ref[...], preferred_element_type=jnp.float32
        )
        hf, cf = _cell(
            gates[:nb] + xwf_ref[j].astype(jnp.float32), h[:nb], c_ref[:nb], hid
        )
        hb, cb = _cell(
            gates[nb:] + xwb_ref[jb].astype(jnp.float32), h[nb:], c_ref[nb:], hid
        )
        h_ref[:nb] = hf
        h_ref[nb:] = hb
        c_ref[:nb] = cf
        c_ref[nb:] = cb
        of_ref[j] = hf.astype(jnp.bfloat16)
        ob_ref[jb] = hb.astype(jnp.bfloat16)


def kernel(inputs, W_ih, W_hh, b_ih, b_hh):
    T, B, I = inputs.shape
    G = W_ih.shape[0]
    hid = G // 4
    ts = _TS
    nt = T // ts

    x2d = inputs.reshape(T * B, I)
    bias = (b_ih + b_hh).reshape(1, G)
    bm = 1024
    xw = pl.pallas_call(
        _proj_kernel,
        grid=((T * B) // bm,),
        in_specs=[
            pl.BlockSpec((bm, I), lambda m: (m, 0)),
            pl.BlockSpec((I, G), lambda m: (0, 0)),
            pl.BlockSpec((1, G), lambda m: (0, 0)),
        ],
        out_specs=pl.BlockSpec((bm, G), lambda m: (m, 0)),
        out_shape=jax.ShapeDtypeStruct((T * B, G), jnp.bfloat16),
        compiler_params=pltpu.CompilerParams(
            dimension_semantics=("arbitrary",),
            vmem_limit_bytes=48 * 1024 * 1024,
        ),
        name="lstm_in_proj",
    )(x2d, W_ih.T.astype(jnp.bfloat16), bias).reshape(T, B, G)

    out_f, out_b = pl.pallas_call(
        functools.partial(_lstm_kernel, ts=ts, hid=hid, nb=B),
        grid=(nt,),
        in_specs=[
            pl.BlockSpec((ts, B, G), lambda t: (t, 0, 0)),
            pl.BlockSpec((ts, B, G), lambda t: (nt - 1 - t, 0, 0)),
            pl.BlockSpec((hid, G), lambda t: (0, 0)),
        ],
        out_specs=[
            pl.BlockSpec((ts, B, hid), lambda t: (t, 0, 0)),
            pl.BlockSpec((ts, B, hid), lambda t: (nt - 1 - t, 0, 0)),
        ],
        out_shape=[
            jax.ShapeDtypeStruct((T, B, hid), jnp.bfloat16),
            jax.ShapeDtypeStruct((T, B, hid), jnp.bfloat16),
        ],
        scratch_shapes=[
            pltpu.VMEM((2 * B, hid), jnp.float32),
            pltpu.VMEM((2 * B, hid), jnp.float32),
        ],
        compiler_params=pltpu.CompilerParams(
            dimension_semantics=("arbitrary",),
            vmem_limit_bytes=48 * 1024 * 1024,
        ),
        name="lstm_recurrence",
    )(xw, xw, W_hh.T.astype(jnp.bfloat16))

    return out_f.astype(jnp.float32) + out_b.astype(jnp.float32)


# speedup vs baseline: 1.1107x; 1.0037x over previous
import functools

import jax
import jax.numpy as jnp
from jax.experimental import pallas as pl
from jax.experimental.pallas import tpu as pltpu

_Z_CELL = 0.1
_Z_HID = 0.1
_TS = 16


def _sigmoid(x):
    return 0.5 + 0.5 * jnp.tanh(0.5 * x)


def _proj_kernel(x_ref, w_ref, b_ref, o_ref):
    o_ref[...] = (
        jnp.dot(
            x_ref[...].astype(jnp.bfloat16),
            w_ref[...],
            preferred_element_type=jnp.float32,
        )
        + b_ref[...]
    ).astype(jnp.bfloat16)


def _cell(xw, h, c, hid):
    gi = _sigmoid(xw[:, :hid])
    gf = _sigmoid(xw[:, hid : 2 * hid])
    gg = jnp.tanh(xw[:, 2 * hid : 3 * hid])
    go = _sigmoid(xw[:, 3 * hid :])
    c_new = gf * c + gi * gg
    h_new = go * jnp.tanh(c_new)
    c_out = (1.0 - _Z_CELL) * c_new + _Z_CELL * c
    h_out = (1.0 - _Z_HID) * h_new + _Z_HID * h
    return h_out, c_out


def _lstm_kernel(xwf_ref, xwb_ref, whh_ref, of_ref, ob_ref, h_ref, c_ref, *, ts, hid, nb):
    @pl.when(pl.program_id(0) == 0)
    def _():
        h_ref[...] = jnp.zeros_like(h_ref)
        c_ref[...] = jnp.zeros_like(c_ref)

    for j in range(ts):
        jb = ts - 1 - j
        h = h_ref[...]
        gates = jnp.dot(
            h.astype(jnp.bfloat16), whh_ref[...], preferred_element_type=jnp.float32
        )
        hf, cf = _cell(
            gates[:nb] + xwf_ref[j].astype(jnp.float32), h[:nb], c_ref[:nb], hid
        )
        hb, cb = _cell(
            gates[nb:] + xwb_ref[jb].astype(jnp.float32), h[nb:], c_ref[nb:], hid
        )
        h_ref[:nb] = hf
        h_ref[nb:] = hb
        c_ref[:nb] = cf
        c_ref[nb:] = cb
        of_ref[j] = hf.astype(jnp.bfloat16)
        ob_ref[jb] = hb.astype(jnp.bfloat16)


def kernel(inputs, W_ih, W_hh, b_ih, b_hh):
    T, B, I = inputs.shape
    G = W_ih.shape[0]
    hid = G // 4
    ts = _TS
    nt = T // ts

    x2d = inputs.reshape(T * B, I)
    bias = (b_ih + b_hh).reshape(1, G)
    bm = 1024
    xw = pl.pallas_call(
        _proj_kernel,
        grid=((T * B) // bm,),
        in_specs=[
            pl.BlockSpec((bm, I), lambda m: (m, 0)),
            pl.BlockSpec((I, G), lambda m: (0, 0)),
            pl.BlockSpec((1, G), lambda m: (0, 0)),
        ],
        out_specs=pl.BlockSpec((bm, G), lambda m: (m, 0)),
        out_shape=jax.ShapeDtypeStruct((T * B, G), jnp.bfloat16),
        compiler_params=pltpu.CompilerParams(
            dimension_semantics=("arbitrary",),
            vmem_limit_bytes=48 * 1024 * 1024,
        ),
        name="lstm_in_proj",
    )(x2d, W_ih.T.astype(jnp.bfloat16), bias).reshape(T, B, G)

    out_f, out_b = pl.pallas_call(
        functools.partial(_lstm_kernel, ts=ts, hid=hid, nb=B),
        grid=(nt,),
        in_specs=[
            pl.BlockSpec((ts, B, G), lambda t: (t, 0, 0)),
            pl.BlockSpec((ts, B, G), lambda t: (nt - 1 - t, 0, 0)),
            pl.BlockSpec((hid, G), lambda t: (0, 0)),
        ],
        out_specs=[
            pl.BlockSpec((ts, B, hid), lambda t: (t, 0, 0)),
            pl.BlockSpec((ts, B, hid), lambda t: (nt - 1 - t, 0, 0)),
        ],
        out_shape=[
            jax.ShapeDtypeStruct((T, B, hid), jnp.bfloat16),
            jax.ShapeDtypeStruct((T, B, hid), jnp.bfloat16),
        ],
        scratch_shapes=[
            pltpu.VMEM((2 * B, hid), jnp.float32),
            pltpu.VMEM((2 * B, hid), jnp.float32),
        ],
        compiler_params=pltpu.CompilerParams(
            dimension_semantics=("arbitrary",),
            vmem_limit_bytes=48 * 1024 * 1024,
        ),
        name="lstm_recurrence",
    )(xw, xw, W_hh.T.astype(jnp.bfloat16))

    return out_f.astype(jnp.float32) + out_b.astype(jnp.float32)


# speedup vs baseline: 1.1173x; 1.0060x over previous
import functools

import jax
import jax.numpy as jnp
from jax.experimental import pallas as pl
from jax.experimental.pallas import tpu as pltpu

_Z_CELL = 0.1
_Z_HID = 0.1
_TS = 16


def _sigmoid(x):
    return 0.5 + 0.5 * jnp.tanh(0.5 * x)


def _proj_kernel(x_ref, w_ref, b_ref, o_ref):
    o_ref[...] = (
        jnp.dot(
            x_ref[...].astype(jnp.bfloat16),
            w_ref[...],
            preferred_element_type=jnp.float32,
        )
        + b_ref[...]
    ).astype(jnp.bfloat16)


def _cell(xw, h, c, hid):
    gi = _sigmoid(xw[:, :hid])
    gf = _sigmoid(xw[:, hid : 2 * hid])
    gg = jnp.tanh(xw[:, 2 * hid : 3 * hid])
    go = _sigmoid(xw[:, 3 * hid :])
    c_new = gf * c + gi * gg
    h_new = go * jnp.tanh(c_new)
    c_out = (1.0 - _Z_CELL) * c_new + _Z_CELL * c
    h_out = (1.0 - _Z_HID) * h_new + _Z_HID * h
    return h_out, c_out


def _lstm_kernel(xwf_ref, xwb_ref, whh_ref, of_ref, ob_ref, h_ref, c_ref, *, ts, hid, nb):
    @pl.when(pl.program_id(0) == 0)
    def _():
        h_ref[...] = jnp.zeros_like(h_ref)
        c_ref[...] = jnp.zeros_like(c_ref)

    for j in range(ts):
        jb = ts - 1 - j
        h = h_ref[...]
        gates = jnp.dot(
            h.astype(jnp.bfloat16), whh_ref[...], preferred_element_type=jnp.float32
        )
        hf, cf = _cell(
            gates[:nb] + xwf_ref[j].astype(jnp.float32), h[:nb], c_ref[:nb], hid
        )
        hb, cb = _cell(
            gates[nb:] + xwb_ref[jb].astype(jnp.float32), h[nb:], c_ref[nb:], hid
        )
        h_ref[:nb] = hf
        h_ref[nb:] = hb
        c_ref[:nb] = cf
        c_ref[nb:] = cb
        of_ref[j] = hf.astype(jnp.bfloat16)
        ob_ref[jb] = hb.astype(jnp.bfloat16)


def kernel(inputs, W_ih, W_hh, b_ih, b_hh):
    T, B, I = inputs.shape
    G = W_ih.shape[0]
    hid = G // 4
    ts = _TS
    nt = T // ts

    x2d = inputs.reshape(T * B, I)
    bias = (b_ih + b_hh).reshape(1, G)
    bm = 2048
    xw = pl.pallas_call(
        _proj_kernel,
        grid=((T * B) // bm,),
        in_specs=[
            pl.BlockSpec((bm, I), lambda m: (m, 0)),
            pl.BlockSpec((I, G), lambda m: (0, 0)),
            pl.BlockSpec((1, G), lambda m: (0, 0)),
        ],
        out_specs=pl.BlockSpec((bm, G), lambda m: (m, 0)),
        out_shape=jax.ShapeDtypeStruct((T * B, G), jnp.bfloat16),
        compiler_params=pltpu.CompilerParams(
            dimension_semantics=("arbitrary",),
            vmem_limit_bytes=48 * 1024 * 1024,
        ),
        name="lstm_in_proj",
    )(x2d, W_ih.T.astype(jnp.bfloat16), bias).reshape(T, B, G)

    out_f, out_b = pl.pallas_call(
        functools.partial(_lstm_kernel, ts=ts, hid=hid, nb=B),
        grid=(nt,),
        in_specs=[
            pl.BlockSpec((ts, B, G), lambda t: (t, 0, 0)),
            pl.BlockSpec((ts, B, G), lambda t: (nt - 1 - t, 0, 0)),
            pl.BlockSpec((hid, G), lambda t: (0, 0)),
        ],
        out_specs=[
            pl.BlockSpec((ts, B, hid), lambda t: (t, 0, 0)),
            pl.BlockSpec((ts, B, hid), lambda t: (nt - 1 - t, 0, 0)),
        ],
        out_shape=[
            jax.ShapeDtypeStruct((T, B, hid), jnp.bfloat16),
            jax.ShapeDtypeStruct((T, B, hid), jnp.bfloat16),
        ],
        scratch_shapes=[
            pltpu.VMEM((2 * B, hid), jnp.float32),
            pltpu.VMEM((2 * B, hid), jnp.float32),
        ],
        compiler_params=pltpu.CompilerParams(
            dimension_semantics=("arbitrary",),
            vmem_limit_bytes=48 * 1024 * 1024,
        ),
        name="lstm_recurrence",
    )(xw, xw, W_hh.T.astype(jnp.bfloat16))

    return out_f.astype(jnp.float32) + out_b.astype(jnp.float32)
